# baseline (speedup 1.0000x reference)
import functools

import jax
import jax.numpy as jnp
from jax import lax
from jax.experimental import pallas as pl
from jax.experimental.pallas import tpu as pltpu
from jax.experimental.pallas import tpu_sc as plsc

B = 16384
D = 64
DP = 128
N_W = 32
BPW = B // N_W
N_PASS = 2
BPP = BPW // N_PASS
G = 16
N_REL_PHYS = 50


def _make_sc_kernel():
    mesh = plsc.VectorSubcoreMesh(core_axis_name="c", subcore_axis_name="s")

    @functools.partial(
        pl.kernel,
        mesh=mesh,
        out_type=jax.ShapeDtypeStruct((D, B), jnp.float32),
        compiler_params=pltpu.CompilerParams(
            use_tc_tiling_on_sc=True, needs_layout_passes=False),
        scratch_types=[
            pltpu.VMEM((BPP,), jnp.int32),
            pltpu.VMEM((BPP,), jnp.int32),
            pltpu.VMEM((BPP,), jnp.int32),
            pltpu.VMEM((BPP,), jnp.int32),
            pltpu.VMEM((BPP,), jnp.int32),
            pltpu.VMEM((BPP, DP), jnp.float32),
            pltpu.VMEM((BPP, DP), jnp.float32),
            pltpu.VMEM((N_REL_PHYS, DP), jnp.float32),
            pltpu.VMEM((D, BPP), jnp.float32),
            pltpu.SemaphoreType.DMA,
            pltpu.SemaphoreType.DMA,
        ],
    )
    def sc_kernel(pred_hbm, rel_hbm, h_hbm, t_hbm, r_hbm, out_hbm,
                  hidx, tidx, ridx, hphys, tphys, hb, tb, relb, obt,
                  sem_h, sem_t):
        wid = lax.axis_index("s") * 2 + lax.axis_index("c")
        base = wid * BPW
        pltpu.sync_copy(rel_hbm, relb)
        lanes = lax.iota(jnp.int32, 16)

        for p in range(N_PASS):
            off = base + p * BPP
            pltpu.sync_copy(h_hbm.at[pl.ds(off, BPP)], hidx)
            pltpu.sync_copy(t_hbm.at[pl.ds(off, BPP)], tidx)
            pltpu.sync_copy(r_hbm.at[pl.ds(off, BPP)], ridx)

            def to_phys(g, carry):
                s = pl.ds(g * 16, 16)
                hphys[s] = lax.shift_right_logical(hidx[s], 1)
                tphys[s] = lax.shift_right_logical(tidx[s], 1)
                return carry

            lax.fori_loop(0, BPP // 16, to_phys, 0)

            ch = pltpu.async_copy(pred_hbm.at[hphys], hb, sem_h)
            ct = pltpu.async_copy(pred_hbm.at[tphys], tb, sem_t)
            ch.wait()
            ct.wait()

            def group(g, carry):
                s = pl.ds(g * 16, 16)
                hv = hidx[s]
                tv = tidx[s]
                rv = ridx[s]
                hcol = lax.shift_left((hv & 1), 6)
                tcol = lax.shift_left((tv & 1), 6)
                rrow = lax.shift_right_logical(rv, 1)
                rcol = lax.shift_left((rv & 1), 6)
                rowv = g * 16 + lanes
                for d in range(D):
                    hval = plsc.load_gather(hb, [rowv, hcol + d])
                    tval = plsc.load_gather(tb, [rowv, tcol + d])
                    rval = plsc.load_gather(relb, [rrow, rcol + d])
                    obt[d, s] = jnp.abs(hval + rval - tval)
                return carry

            lax.fori_loop(0, BPP // G, group, 0)

            pltpu.sync_copy(obt, out_hbm.at[:, pl.ds(off, BPP)])

    return sc_kernel


def kernel(pred_table, rel_table, h_idx, t_idx, r_idx):
    n_preds, n_rel = pred_table.shape[0], rel_table.shape[0]
    pred2 = jax.lax.bitcast_convert_type(pred_table, jnp.uint16)
    pred2 = jax.lax.bitcast_convert_type(
        pred2.reshape(n_preds // 2, DP, 2), jnp.float32)
    rel2 = jax.lax.bitcast_convert_type(rel_table, jnp.uint16)
    rel2 = jax.lax.bitcast_convert_type(
        rel2.reshape(n_rel // 2, DP, 2), jnp.float32)
    sc = _make_sc_kernel()
    out_t = sc(
        pred2,
        rel2,
        h_idx.astype(jnp.int32),
        t_idx.astype(jnp.int32),
        r_idx.astype(jnp.int32),
    )
    return out_t.T


# speedup vs baseline: 1.7096x; 1.7096x over previous
import functools

import jax
import jax.numpy as jnp
from jax import lax
from jax.experimental import pallas as pl
from jax.experimental.pallas import tpu as pltpu
from jax.experimental.pallas import tpu_sc as plsc

B = 16384
D = 64
DP = 128
N_W = 32
BPW = B // N_W
N_PASS = 2
BPP = BPW // N_PASS
G = 16
N_REL_PHYS = 50


def _make_sc_kernel():
    mesh = plsc.VectorSubcoreMesh(core_axis_name="c", subcore_axis_name="s")

    @functools.partial(
        pl.kernel,
        mesh=mesh,
        out_type=jax.ShapeDtypeStruct((D, B), jnp.float32),
        compiler_params=pltpu.CompilerParams(
            use_tc_tiling_on_sc=True, needs_layout_passes=False),
        scratch_types=[
            pltpu.VMEM((BPP,), jnp.int32),
            pltpu.VMEM((BPP,), jnp.int32),
            pltpu.VMEM((BPP,), jnp.int32),
            pltpu.VMEM((BPP,), jnp.int32),
            pltpu.VMEM((BPP,), jnp.int32),
            pltpu.VMEM((BPP, DP), jnp.float32),
            pltpu.VMEM((BPP, DP), jnp.float32),
            pltpu.VMEM((N_REL_PHYS, DP), jnp.float32),
            pltpu.VMEM((D, BPP), jnp.float32),
            pltpu.SemaphoreType.DMA,
            pltpu.SemaphoreType.DMA,
        ],
    )
    def sc_kernel(pred_hbm, rel_hbm, h_hbm, t_hbm, r_hbm, out_hbm,
                  hidx, tidx, ridx, hphys, tphys, hb, tb, relb, obt,
                  sem_h, sem_t):
        wid = lax.axis_index("s") * 2 + lax.axis_index("c")
        base = wid * BPW
        pltpu.sync_copy(rel_hbm, relb)
        lanes = lax.iota(jnp.int32, 16)

        for p in range(N_PASS):
            off = base + p * BPP
            pltpu.sync_copy(h_hbm.at[pl.ds(off, BPP)], hidx)
            pltpu.sync_copy(t_hbm.at[pl.ds(off, BPP)], tidx)
            pltpu.sync_copy(r_hbm.at[pl.ds(off, BPP)], ridx)

            def to_phys(g, carry):
                s = pl.ds(g * 16, 16)
                hphys[s] = lax.shift_right_logical(hidx[s], 1)
                tphys[s] = lax.shift_right_logical(tidx[s], 1)
                return carry

            lax.fori_loop(0, BPP // 16, to_phys, 0)

            ch = pltpu.async_copy(pred_hbm.at[hphys], hb, sem_h)
            ct = pltpu.async_copy(pred_hbm.at[tphys], tb, sem_t)
            ch.wait()
            ct.wait()

            def group(g, carry):
                s = pl.ds(g * 16, 16)
                hv = hidx[s]
                tv = tidx[s]
                rv = ridx[s]
                hcol = lax.shift_left((hv & 1), 6)
                tcol = lax.shift_left((tv & 1), 6)
                rrow = lax.shift_right_logical(rv, 1)
                rcol = lax.shift_left((rv & 1), 6)
                rowv = g * 16 + lanes
                for d in range(D):
                    hval = plsc.load_gather(hb, [rowv, hcol + d])
                    tval = plsc.load_gather(tb, [rowv, tcol + d])
                    rval = plsc.load_gather(relb, [rrow, rcol + d])
                    obt[d, s] = jnp.abs(hval + rval - tval)
                return carry

            lax.fori_loop(0, BPP // G, group, 0)

            pltpu.sync_copy(obt, out_hbm.at[:, pl.ds(off, BPP)])

    return sc_kernel


def kernel(pred_table, rel_table, h_idx, t_idx, r_idx):
    n_preds, n_rel = pred_table.shape[0], rel_table.shape[0]
    pred2 = pred_table.reshape(n_preds // 2, DP)
    rel2 = rel_table.reshape(n_rel // 2, DP)
    sc = _make_sc_kernel()
    out_t = sc(
        pred2,
        rel2,
        h_idx.astype(jnp.int32),
        t_idx.astype(jnp.int32),
        r_idx.astype(jnp.int32),
    )
    return out_t.T


# speedup vs baseline: 2.1384x; 1.2508x over previous
import functools

import jax
import jax.numpy as jnp
from jax import lax
from jax.experimental import pallas as pl
from jax.experimental.pallas import tpu as pltpu
from jax.experimental.pallas import tpu_sc as plsc

B = 16384
D = 64
DP = 128
N_W = 32
BPW = B // N_W
N_PASS = 2
BPP = BPW // N_PASS
G = 16
N_PREDS = 1000000
HALF = N_PREDS // 2
TC_COLS = 2048
TC_ROWS = TC_COLS // 2
TC_GRID = -(-N_PREDS // TC_COLS)
N_REL_PHYS = 50


def _tc_pack(pred_t, eye):
    def body(x_ref, eye_ref, o_ref):
        dn = (((0,), (0,)), ((), ()))
        xt = lax.dot_general(x_ref[...], eye_ref[...], dn,
                             preferred_element_type=jnp.float32)
        v = xt.reshape(TC_COLS // 16, 16, D)
        ya = v[:, :8, :].reshape(TC_ROWS, D)
        yb = v[:, 8:, :].reshape(TC_ROWS, D)
        o_ref[...] = jnp.concatenate([ya, yb], axis=1)

    return pl.pallas_call(
        body,
        grid=(TC_GRID,),
        in_specs=[
            pl.BlockSpec((D, TC_COLS), lambda i: (0, i)),
            pl.BlockSpec((D, D), lambda i: (0, 0)),
        ],
        out_specs=pl.BlockSpec((TC_ROWS, DP), lambda i: (i, 0)),
        out_shape=jax.ShapeDtypeStruct((HALF, DP), jnp.float32),
    )(pred_t, eye)


def _make_sc_kernel():
    mesh = plsc.VectorSubcoreMesh(core_axis_name="c", subcore_axis_name="s")

    @functools.partial(
        pl.kernel,
        mesh=mesh,
        out_type=jax.ShapeDtypeStruct((D, B), jnp.float32),
        compiler_params=pltpu.CompilerParams(
            use_tc_tiling_on_sc=True, needs_layout_passes=False),
        scratch_types=[
            pltpu.VMEM((BPP,), jnp.int32),
            pltpu.VMEM((BPP,), jnp.int32),
            pltpu.VMEM((BPP,), jnp.int32),
            pltpu.VMEM((BPP,), jnp.int32),
            pltpu.VMEM((BPP,), jnp.int32),
            pltpu.VMEM((BPP, DP), jnp.float32),
            pltpu.VMEM((BPP, DP), jnp.float32),
            pltpu.VMEM((N_REL_PHYS, DP), jnp.float32),
            pltpu.VMEM((D, BPP), jnp.float32),
            pltpu.SemaphoreType.DMA,
            pltpu.SemaphoreType.DMA,
        ],
    )
    def sc_kernel(pred_hbm, rel_hbm, h_hbm, t_hbm, r_hbm, out_hbm,
                  hidx, tidx, ridx, hphys, tphys, hb, tb, relb, obt,
                  sem_h, sem_t):
        wid = lax.axis_index("s") * 2 + lax.axis_index("c")
        base = wid * BPW
        pltpu.sync_copy(rel_hbm, relb)
        lanes = lax.iota(jnp.int32, 16)

        for p in range(N_PASS):
            off = base + p * BPP
            pltpu.sync_copy(h_hbm.at[pl.ds(off, BPP)], hidx)
            pltpu.sync_copy(t_hbm.at[pl.ds(off, BPP)], tidx)
            pltpu.sync_copy(r_hbm.at[pl.ds(off, BPP)], ridx)

            def to_phys(g, carry):
                s = pl.ds(g * 16, 16)
                hv = hidx[s]
                tv = tidx[s]
                hphys[s] = lax.shift_left(lax.shift_right_logical(hv, 4), 3) | (hv & 7)
                tphys[s] = lax.shift_left(lax.shift_right_logical(tv, 4), 3) | (tv & 7)
                return carry

            lax.fori_loop(0, BPP // 16, to_phys, 0)

            ch = pltpu.async_copy(pred_hbm.at[hphys], hb, sem_h)
            ct = pltpu.async_copy(pred_hbm.at[tphys], tb, sem_t)
            ch.wait()
            ct.wait()

            def group(g, carry):
                s = pl.ds(g * 16, 16)
                hv = hidx[s]
                tv = tidx[s]
                rv = ridx[s]
                hcol = lax.shift_left(hv & 8, 3)
                tcol = lax.shift_left(tv & 8, 3)
                rrow = lax.shift_right_logical(rv, 1)
                rcol = lax.shift_left((rv & 1), 6)
                rowv = g * 16 + lanes
                for d in range(D):
                    hval = plsc.load_gather(hb, [rowv, hcol + d])
                    tval = plsc.load_gather(tb, [rowv, tcol + d])
                    rval = plsc.load_gather(relb, [rrow, rcol + d])
                    obt[d, s] = jnp.abs(hval + rval - tval)
                return carry

            lax.fori_loop(0, BPP // G, group, 0)

            pltpu.sync_copy(obt, out_hbm.at[:, pl.ds(off, BPP)])

    return sc_kernel


def kernel(pred_table, rel_table, h_idx, t_idx, r_idx):
    n_rel = rel_table.shape[0]
    eye = jnp.eye(D, dtype=jnp.float32)
    pred2 = _tc_pack(pred_table.T, eye)
    rel2 = rel_table.reshape(n_rel // 2, DP)
    sc = _make_sc_kernel()
    out_t = sc(
        pred2,
        rel2,
        h_idx.astype(jnp.int32),
        t_idx.astype(jnp.int32),
        r_idx.astype(jnp.int32),
    )
    return out_t.T


# speedup vs baseline: 2.8193x; 1.3185x over previous
import functools

import jax
import jax.numpy as jnp
from jax import lax
from jax.experimental import pallas as pl
from jax.experimental.pallas import tpu as pltpu
from jax.experimental.pallas import tpu_sc as plsc

B = 16384
D = 64
DP = 128
N_W = 32
BPW = B // N_W
N_PASS = 2
BPP = BPW // N_PASS
G = 16
N_PREDS = 1000000
HALF = N_PREDS // 2
TC_COLS = 4096
TC_ROWS = TC_COLS // 2
TC_GRID = -(-N_PREDS // TC_COLS)
N_REL_PHYS = 50


def _tc_pack(pred_t):
    def body(x_ref, o_ref):
        xt = lax.transpose(x_ref[...], (1, 0))
        v = xt.reshape(TC_COLS // 16, 16, D)
        ya = v[:, :8, :].reshape(TC_ROWS, D)
        yb = v[:, 8:, :].reshape(TC_ROWS, D)
        o_ref[...] = jnp.concatenate([ya, yb], axis=1)

    return pl.pallas_call(
        body,
        grid=(TC_GRID,),
        in_specs=[
            pl.BlockSpec((D, TC_COLS), lambda i: (0, i)),
        ],
        out_specs=pl.BlockSpec((TC_ROWS, DP), lambda i: (i, 0)),
        out_shape=jax.ShapeDtypeStruct((HALF, DP), jnp.float32),
    )(pred_t)


def _make_sc_kernel():
    mesh = plsc.VectorSubcoreMesh(core_axis_name="c", subcore_axis_name="s")

    @functools.partial(
        pl.kernel,
        mesh=mesh,
        out_type=jax.ShapeDtypeStruct((D, B), jnp.float32),
        compiler_params=pltpu.CompilerParams(
            use_tc_tiling_on_sc=True, needs_layout_passes=False),
        scratch_types=[
            pltpu.VMEM((BPP,), jnp.int32),
            pltpu.VMEM((BPP,), jnp.int32),
            pltpu.VMEM((BPP,), jnp.int32),
            pltpu.VMEM((BPP,), jnp.int32),
            pltpu.VMEM((BPP,), jnp.int32),
            pltpu.VMEM((BPP, DP), jnp.float32),
            pltpu.VMEM((BPP, DP), jnp.float32),
            pltpu.VMEM((N_REL_PHYS, DP), jnp.float32),
            pltpu.VMEM((D, BPP), jnp.float32),
            pltpu.SemaphoreType.DMA,
            pltpu.SemaphoreType.DMA,
        ],
    )
    def sc_kernel(pred_hbm, rel_hbm, h_hbm, t_hbm, r_hbm, out_hbm,
                  hidx, tidx, ridx, hphys, tphys, hb, tb, relb, obt,
                  sem_h, sem_t):
        wid = lax.axis_index("s") * 2 + lax.axis_index("c")
        base = wid * BPW
        pltpu.sync_copy(rel_hbm, relb)
        lanes = lax.iota(jnp.int32, 16)

        for p in range(N_PASS):
            off = base + p * BPP
            pltpu.sync_copy(h_hbm.at[pl.ds(off, BPP)], hidx)
            pltpu.sync_copy(t_hbm.at[pl.ds(off, BPP)], tidx)
            pltpu.sync_copy(r_hbm.at[pl.ds(off, BPP)], ridx)

            def to_phys(g, carry):
                s = pl.ds(g * 16, 16)
                hv = hidx[s]
                tv = tidx[s]
                hphys[s] = lax.shift_left(lax.shift_right_logical(hv, 4), 3) | (hv & 7)
                tphys[s] = lax.shift_left(lax.shift_right_logical(tv, 4), 3) | (tv & 7)
                return carry

            lax.fori_loop(0, BPP // 16, to_phys, 0)

            ch = pltpu.async_copy(pred_hbm.at[hphys], hb, sem_h)
            ct = pltpu.async_copy(pred_hbm.at[tphys], tb, sem_t)
            ch.wait()
            ct.wait()

            def group(g, carry):
                s = pl.ds(g * 16, 16)
                hv = hidx[s]
                tv = tidx[s]
                rv = ridx[s]
                hcol = lax.shift_left(hv & 8, 3)
                tcol = lax.shift_left(tv & 8, 3)
                rrow = lax.shift_right_logical(rv, 1)
                rcol = lax.shift_left((rv & 1), 6)
                rowv = g * 16 + lanes
                for d in range(D):
                    hval = plsc.load_gather(hb, [rowv, hcol + d])
                    tval = plsc.load_gather(tb, [rowv, tcol + d])
                    rval = plsc.load_gather(relb, [rrow, rcol + d])
                    obt[d, s] = jnp.abs(hval + rval - tval)
                return carry

            lax.fori_loop(0, BPP // G, group, 0)

            pltpu.sync_copy(obt, out_hbm.at[:, pl.ds(off, BPP)])

    return sc_kernel


def kernel(pred_table, rel_table, h_idx, t_idx, r_idx):
    n_rel = rel_table.shape[0]
    pred2 = _tc_pack(pred_table.T)
    rel2 = rel_table.reshape(n_rel // 2, DP)
    sc = _make_sc_kernel()
    out_t = sc(
        pred2,
        rel2,
        h_idx.astype(jnp.int32),
        t_idx.astype(jnp.int32),
        r_idx.astype(jnp.int32),
    )
    return out_t.T


# speedup vs baseline: 3.3659x; 1.1939x over previous
import functools

import jax
import jax.numpy as jnp
from jax import lax
from jax.experimental import pallas as pl
from jax.experimental.pallas import tpu as pltpu
from jax.experimental.pallas import tpu_sc as plsc

B = 16384
D = 64
DP = 128
N_W = 32
BPW = B // N_W
N_PASS = 2
BPP = BPW // N_PASS
G = 16
N_PREDS = 1000000
HALF = N_PREDS // 2
TC_COLS = 8192
TC_ROWS = TC_COLS // 2
TC_GRID = -(-N_PREDS // TC_COLS)
N_REL_PHYS = 50


def _tc_pack(pred_t):
    def body(x_ref, o_ref):
        xt = lax.transpose(x_ref[...], (1, 0))
        v = xt.reshape(TC_COLS // 16, 16, D)
        ya = v[:, :8, :].reshape(TC_ROWS, D)
        yb = v[:, 8:, :].reshape(TC_ROWS, D)
        o_ref[...] = jnp.concatenate([ya, yb], axis=1)

    return pl.pallas_call(
        body,
        grid=(TC_GRID,),
        in_specs=[
            pl.BlockSpec((D, TC_COLS), lambda i: (0, i)),
        ],
        out_specs=pl.BlockSpec((TC_ROWS, DP), lambda i: (i, 0)),
        out_shape=jax.ShapeDtypeStruct((HALF, DP), jnp.float32),
    )(pred_t)


def _make_sc_kernel():
    mesh = plsc.VectorSubcoreMesh(core_axis_name="c", subcore_axis_name="s")

    @functools.partial(
        pl.kernel,
        mesh=mesh,
        out_type=jax.ShapeDtypeStruct((D, B), jnp.float32),
        compiler_params=pltpu.CompilerParams(
            use_tc_tiling_on_sc=True, needs_layout_passes=False),
        scratch_types=[
            pltpu.VMEM((BPP,), jnp.int32),
            pltpu.VMEM((BPP,), jnp.int32),
            pltpu.VMEM((BPP,), jnp.int32),
            pltpu.VMEM((BPP,), jnp.int32),
            pltpu.VMEM((BPP,), jnp.int32),
            pltpu.VMEM((BPP, DP), jnp.float32),
            pltpu.VMEM((BPP, DP), jnp.float32),
            pltpu.VMEM((N_REL_PHYS, DP), jnp.float32),
            pltpu.VMEM((D, BPP), jnp.float32),
            pltpu.SemaphoreType.DMA,
            pltpu.SemaphoreType.DMA,
        ],
    )
    def sc_kernel(pred_hbm, rel_hbm, h_hbm, t_hbm, r_hbm, out_hbm,
                  hidx, tidx, ridx, hphys, tphys, hb, tb, relb, obt,
                  sem_h, sem_t):
        wid = lax.axis_index("s") * 2 + lax.axis_index("c")
        base = wid * BPW
        pltpu.sync_copy(rel_hbm, relb)
        lanes = lax.iota(jnp.int32, 16)

        for p in range(N_PASS):
            off = base + p * BPP
            pltpu.sync_copy(h_hbm.at[pl.ds(off, BPP)], hidx)
            pltpu.sync_copy(t_hbm.at[pl.ds(off, BPP)], tidx)
            pltpu.sync_copy(r_hbm.at[pl.ds(off, BPP)], ridx)

            def to_phys(g, carry):
                s = pl.ds(g * 16, 16)
                hv = hidx[s]
                tv = tidx[s]
                hphys[s] = lax.shift_left(lax.shift_right_logical(hv, 4), 3) | (hv & 7)
                tphys[s] = lax.shift_left(lax.shift_right_logical(tv, 4), 3) | (tv & 7)
                return carry

            lax.fori_loop(0, BPP // 16, to_phys, 0)

            ch = pltpu.async_copy(pred_hbm.at[hphys], hb, sem_h)
            ct = pltpu.async_copy(pred_hbm.at[tphys], tb, sem_t)
            ch.wait()
            ct.wait()

            def group(g, carry):
                s = pl.ds(g * 16, 16)
                hv = hidx[s]
                tv = tidx[s]
                rv = ridx[s]
                hcol = lax.shift_left(hv & 8, 3)
                tcol = lax.shift_left(tv & 8, 3)
                rrow = lax.shift_right_logical(rv, 1)
                rcol = lax.shift_left((rv & 1), 6)
                rowv = g * 16 + lanes
                for d in range(D):
                    hval = plsc.load_gather(hb, [rowv, hcol + d])
                    tval = plsc.load_gather(tb, [rowv, tcol + d])
                    rval = plsc.load_gather(relb, [rrow, rcol + d])
                    obt[d, s] = jnp.abs(hval + rval - tval)
                return carry

            lax.fori_loop(0, BPP // G, group, 0)

            pltpu.sync_copy(obt, out_hbm.at[:, pl.ds(off, BPP)])

    return sc_kernel


def kernel(pred_table, rel_table, h_idx, t_idx, r_idx):
    n_rel = rel_table.shape[0]
    pred2 = _tc_pack(pred_table.T)
    rel2 = rel_table.reshape(n_rel // 2, DP)
    sc = _make_sc_kernel()
    out_t = sc(
        pred2,
        rel2,
        h_idx.astype(jnp.int32),
        t_idx.astype(jnp.int32),
        r_idx.astype(jnp.int32),
    )
    return out_t.T


# speedup vs baseline: 3.7112x; 1.1026x over previous
import functools

import jax
import jax.numpy as jnp
from jax import lax
from jax.experimental import pallas as pl
from jax.experimental.pallas import tpu as pltpu
from jax.experimental.pallas import tpu_sc as plsc

B = 16384
D = 64
DP = 128
N_W = 32
BPW = B // N_W
N_PASS = 2
BPP = BPW // N_PASS
G = 16
N_PREDS = 1000000
HALF = N_PREDS // 2
TC_COLS = 16384
TC_ROWS = TC_COLS // 2
TC_GRID = -(-N_PREDS // TC_COLS)
N_REL_PHYS = 50


def _tc_pack(pred_t):
    def body(x_ref, o_ref):
        xt = lax.transpose(x_ref[...], (1, 0))
        v = xt.reshape(TC_COLS // 16, 16, D)
        ya = v[:, :8, :].reshape(TC_ROWS, D)
        yb = v[:, 8:, :].reshape(TC_ROWS, D)
        o_ref[...] = jnp.concatenate([ya, yb], axis=1)

    return pl.pallas_call(
        body,
        grid=(TC_GRID,),
        in_specs=[
            pl.BlockSpec((D, TC_COLS), lambda i: (0, i)),
        ],
        out_specs=pl.BlockSpec((TC_ROWS, DP), lambda i: (i, 0)),
        out_shape=jax.ShapeDtypeStruct((HALF, DP), jnp.float32),
    )(pred_t)


def _make_sc_kernel():
    mesh = plsc.VectorSubcoreMesh(core_axis_name="c", subcore_axis_name="s")

    @functools.partial(
        pl.kernel,
        mesh=mesh,
        out_type=jax.ShapeDtypeStruct((D, B), jnp.float32),
        compiler_params=pltpu.CompilerParams(
            use_tc_tiling_on_sc=True, needs_layout_passes=False),
        scratch_types=[
            pltpu.VMEM((BPP,), jnp.int32),
            pltpu.VMEM((BPP,), jnp.int32),
            pltpu.VMEM((BPP,), jnp.int32),
            pltpu.VMEM((BPP,), jnp.int32),
            pltpu.VMEM((BPP,), jnp.int32),
            pltpu.VMEM((BPP, DP), jnp.float32),
            pltpu.VMEM((BPP, DP), jnp.float32),
            pltpu.VMEM((N_REL_PHYS, DP), jnp.float32),
            pltpu.VMEM((D, BPP), jnp.float32),
            pltpu.SemaphoreType.DMA,
            pltpu.SemaphoreType.DMA,
        ],
    )
    def sc_kernel(pred_hbm, rel_hbm, h_hbm, t_hbm, r_hbm, out_hbm,
                  hidx, tidx, ridx, hphys, tphys, hb, tb, relb, obt,
                  sem_h, sem_t):
        wid = lax.axis_index("s") * 2 + lax.axis_index("c")
        base = wid * BPW
        pltpu.sync_copy(rel_hbm, relb)
        lanes = lax.iota(jnp.int32, 16)

        for p in range(N_PASS):
            off = base + p * BPP
            pltpu.sync_copy(h_hbm.at[pl.ds(off, BPP)], hidx)
            pltpu.sync_copy(t_hbm.at[pl.ds(off, BPP)], tidx)
            pltpu.sync_copy(r_hbm.at[pl.ds(off, BPP)], ridx)

            def to_phys(g, carry):
                s = pl.ds(g * 16, 16)
                hv = hidx[s]
                tv = tidx[s]
                hphys[s] = lax.shift_left(lax.shift_right_logical(hv, 4), 3) | (hv & 7)
                tphys[s] = lax.shift_left(lax.shift_right_logical(tv, 4), 3) | (tv & 7)
                return carry

            lax.fori_loop(0, BPP // 16, to_phys, 0)

            ch = pltpu.async_copy(pred_hbm.at[hphys], hb, sem_h)
            ct = pltpu.async_copy(pred_hbm.at[tphys], tb, sem_t)
            ch.wait()
            ct.wait()

            def group(g, carry):
                s = pl.ds(g * 16, 16)
                hv = hidx[s]
                tv = tidx[s]
                rv = ridx[s]
                hcol = lax.shift_left(hv & 8, 3)
                tcol = lax.shift_left(tv & 8, 3)
                rrow = lax.shift_right_logical(rv, 1)
                rcol = lax.shift_left((rv & 1), 6)
                rowv = g * 16 + lanes
                for d in range(D):
                    hval = plsc.load_gather(hb, [rowv, hcol + d])
                    tval = plsc.load_gather(tb, [rowv, tcol + d])
                    rval = plsc.load_gather(relb, [rrow, rcol + d])
                    obt[d, s] = jnp.abs(hval + rval - tval)
                return carry

            lax.fori_loop(0, BPP // G, group, 0)

            pltpu.sync_copy(obt, out_hbm.at[:, pl.ds(off, BPP)])

    return sc_kernel


def kernel(pred_table, rel_table, h_idx, t_idx, r_idx):
    n_rel = rel_table.shape[0]
    pred2 = _tc_pack(pred_table.T)
    rel2 = rel_table.reshape(n_rel // 2, DP)
    sc = _make_sc_kernel()
    out_t = sc(
        pred2,
        rel2,
        h_idx.astype(jnp.int32),
        t_idx.astype(jnp.int32),
        r_idx.astype(jnp.int32),
    )
    return out_t.T
